# Initial kernel scaffold; baseline (speedup 1.0000x reference)
import jax
import jax.numpy as jnp
from jax.experimental import pallas as pl


def kernel(nodes, neigh_idx, num_sample, features):
    raise NotImplementedError("write your pallas kernel here")


# speedup vs baseline: 3.8574x; 3.8574x over previous
import functools

import jax
import jax.numpy as jnp
from jax import lax
from jax.experimental import pallas as pl
from jax.experimental.pallas import tpu as pltpu
from jax.experimental.pallas import tpu_sc as plsc

_NC = 2
_NS = 16
_NW = _NC * _NS
_LANES = 16


@functools.partial(jax.jit, static_argnums=(2, 3))
def _gather_mean(idx_flat, features, chunk_rows, S):
    N, D = features.shape
    BP = idx_flat.shape[0] // S
    C = chunk_rows
    b_per_w = BP // _NW
    n_chunks = b_per_w // C
    inv_s = jnp.float32(1.0 / S)

    mesh = plsc.VectorSubcoreMesh(
        core_axis_name="c", subcore_axis_name="s",
        num_cores=_NC, num_subcores=_NS,
    )

    @functools.partial(
        pl.kernel,
        mesh=mesh,
        out_type=jax.ShapeDtypeStruct((BP, D), jnp.float32),
        scratch_types=[
            pltpu.VMEM((C * S,), jnp.int32),
            pltpu.VMEM((C * S, D), jnp.float32),
            pltpu.VMEM((C, D), jnp.float32),
            pltpu.SemaphoreType.DMA,
        ],
    )
    def body(idx_hbm, feat_hbm, out_hbm, idx_v, rows_v, out_v, sem):
        wid = lax.axis_index("s") * _NC + lax.axis_index("c")
        base = wid * b_per_w

        def chunk(g, carry):
            row0 = base + g * C
            pltpu.sync_copy(idx_hbm.at[pl.ds(row0 * S, C * S)], idx_v)
            pltpu.async_copy(feat_hbm.at[idx_v], rows_v, sem).wait()

            def one_row(c, carry2):
                r0 = c * S
                for k in range(D // _LANES):
                    col = pl.ds(k * _LANES, _LANES)
                    acc = rows_v[r0, col]
                    for j in range(1, S):
                        acc = acc + rows_v[r0 + j, col]
                    out_v[c, col] = acc * inv_s
                return carry2

            lax.fori_loop(0, C, one_row, 0)
            pltpu.sync_copy(out_v, out_hbm.at[pl.ds(row0, C)])
            return carry

        lax.fori_loop(0, n_chunks, chunk, 0)

    return body(idx_flat, features)


def kernel(nodes, neigh_idx, num_sample, features):
    B, S = neigh_idx.shape
    del nodes, num_sample
    C = 56
    per = _NW * C
    BP = ((B + per - 1) // per) * per
    idx_flat = jnp.reshape(neigh_idx.astype(jnp.int32), (B * S,))
    if BP != B:
        idx_flat = jnp.concatenate(
            [idx_flat, jnp.zeros(((BP - B) * S,), jnp.int32)]
        )
    out = _gather_mean(idx_flat, features, C, S)
    return out[:B]


# speedup vs baseline: 7.2726x; 1.8854x over previous
import functools

import jax
import jax.numpy as jnp
from jax import lax
from jax.experimental import pallas as pl
from jax.experimental.pallas import tpu as pltpu
from jax.experimental.pallas import tpu_sc as plsc

_NC = 2
_NS = 16
_NW = _NC * _NS
_LANES = 16


@functools.partial(jax.jit, static_argnums=(2, 3))
def _gather_mean(idx_r, features, chunk_rows, S):
    N, D = features.shape
    C = chunk_rows
    BP = idx_r.shape[0] // S
    b_per_w = BP // _NW
    n_chunks = b_per_w // C
    inv_s = jnp.float32(1.0 / S)

    mesh = plsc.VectorSubcoreMesh(
        core_axis_name="c", subcore_axis_name="s",
        num_cores=_NC, num_subcores=_NS,
    )

    @functools.partial(
        pl.kernel,
        mesh=mesh,
        out_type=jax.ShapeDtypeStruct((BP, D), jnp.float32),
        scratch_types=[
            pltpu.VMEM((S * C,), jnp.int32),
            pltpu.VMEM((C, D), jnp.float32),
            pltpu.SemaphoreType.DMA,
            pltpu.SemaphoreType.DMA,
        ],
    )
    def body(idx_hbm, feat_hbm, out_hbm, idx_v, acc_v, sem0, sem):
        wid = lax.axis_index("s") * _NC + lax.axis_index("c")
        base = wid * b_per_w

        def chunk(g, carry):
            row0 = base + g * C
            off = (wid * n_chunks + g) * (S * C)
            pltpu.sync_copy(idx_hbm.at[pl.ds(off, S * C)], idx_v)
            pltpu.async_copy(
                feat_hbm.at[idx_v.at[pl.ds(0, C)]], acc_v, sem0
            ).wait()
            cps = [
                pltpu.async_copy(
                    feat_hbm.at[idx_v.at[pl.ds(j * C, C)]], acc_v, sem,
                    add=True,
                )
                for j in range(1, S)
            ]
            for cp in cps:
                cp.wait()

            def scale_row(c, carry2):
                for k in range(D // _LANES):
                    col = pl.ds(k * _LANES, _LANES)
                    acc_v[c, col] = acc_v[c, col] * inv_s
                return carry2

            lax.fori_loop(0, C, scale_row, 0)
            pltpu.sync_copy(acc_v, out_hbm.at[pl.ds(row0, C)])
            return carry

        lax.fori_loop(0, n_chunks, chunk, 0)

    return body(idx_r, features)


def kernel(nodes, neigh_idx, num_sample, features):
    B, S = neigh_idx.shape
    del nodes, num_sample
    C = 392
    per = _NW * C
    BP = ((B + per - 1) // per) * per
    idx = neigh_idx.astype(jnp.int32)
    if BP != B:
        idx = jnp.concatenate([idx, jnp.zeros((BP - B, S), jnp.int32)])
    idx_r = jnp.reshape(
        jnp.transpose(jnp.reshape(idx, (BP // C, C, S)), (0, 2, 1)), (-1,)
    )
    out = _gather_mean(idx_r, features, C, S)
    return out[:B]


# speedup vs baseline: 8.0041x; 1.1006x over previous
import functools

import jax
import jax.numpy as jnp
from jax import lax
from jax.experimental import pallas as pl
from jax.experimental.pallas import tpu as pltpu
from jax.experimental.pallas import tpu_sc as plsc

_NC = 2
_NS = 16
_NW = _NC * _NS
_LANES = 16
_N0 = 18
_N1 = 10


@functools.partial(jax.jit, static_argnums=(2, 3, 4))
def _gather_mean(idx_r, features, B, chunk_rows, S):
    N, D = features.shape
    C = chunk_rows
    BP = (idx_r.shape[0] // S)
    inv_s = jnp.float32(1.0 / S)
    tail = B % C

    mesh = plsc.VectorSubcoreMesh(
        core_axis_name="c", subcore_axis_name="s",
        num_cores=_NC, num_subcores=_NS,
    )

    @functools.partial(
        pl.kernel,
        mesh=mesh,
        out_type=jax.ShapeDtypeStruct((B, D), jnp.float32),
        scratch_types=[
            pltpu.VMEM((S * C,), jnp.int32),
            pltpu.VMEM((C, D), jnp.float32),
            pltpu.SemaphoreType.DMA,
            pltpu.SemaphoreType.DMA,
        ],
    )
    def body(idx_hbm, feat_hbm, out_hbm, idx_v, acc_v, sem0, sem):
        c = lax.axis_index("c")
        s = lax.axis_index("s")
        n_my = jnp.where(c == 0, _N0, _N1)
        chunk0 = jnp.where(c == 0, s * _N0, _NS * _N0 + s * _N1)

        def chunk(g, carry):
            gid = chunk0 + g
            row0 = gid * C
            pltpu.sync_copy(idx_hbm.at[pl.ds(gid * (S * C), S * C)], idx_v)
            pltpu.async_copy(
                feat_hbm.at[idx_v.at[pl.ds(0, C)]], acc_v, sem0
            ).wait()
            cps = [
                pltpu.async_copy(
                    feat_hbm.at[idx_v.at[pl.ds(j * C, C)]], acc_v, sem,
                    add=True,
                )
                for j in range(1, S)
            ]
            for cp in cps:
                cp.wait()

            def scale_row(r, carry2):
                for k in range(D // _LANES):
                    col = pl.ds(k * _LANES, _LANES)
                    acc_v[r, col] = acc_v[r, col] * inv_s
                return carry2

            lax.fori_loop(0, C, scale_row, 0)

            @pl.when(row0 + C <= B)
            def _full():
                pltpu.sync_copy(acc_v, out_hbm.at[pl.ds(row0, C)])

            if tail:
                @pl.when(row0 == B - tail)
                def _partial():
                    pltpu.sync_copy(
                        acc_v.at[pl.ds(0, tail)],
                        out_hbm.at[pl.ds(B - tail, tail)],
                    )
            return carry

        lax.fori_loop(0, n_my, chunk, 0)

    return body(idx_r, features)


def kernel(nodes, neigh_idx, num_sample, features):
    B, S = neigh_idx.shape
    del nodes, num_sample
    C = 112
    per = _NS * (_N0 + _N1) * C
    BP = ((B + per - 1) // per) * per
    idx = neigh_idx.astype(jnp.int32)
    if BP != B:
        idx = jnp.concatenate([idx, jnp.zeros((BP - B, S), jnp.int32)])
    idx_r = jnp.reshape(
        jnp.transpose(jnp.reshape(idx, (BP // C, C, S)), (0, 2, 1)), (-1,)
    )
    return _gather_mean(idx_r, features, B, C, S)
